# baseline (speedup 1.0000x reference)
import jax
import jax.numpy as jnp
from jax import lax
from jax.experimental import pallas as pl
from jax.experimental.pallas import tpu as pltpu
from jax.experimental.pallas import tpu_sc as plsc

N = 10000
E = 320000
F_IN = 128
H = 64
C = 10
G = 64

NC = 2
NS = 16
NW = NC * NS
EPW = E // NW
K = 128
NCHUNK = -(-EPW // K)
EPWP = NCHUNK * K
NPAD = N + 16
RPT_A = 640
RPT_B = N - (NS - 1) * RPT_A
LANES = 16
NBUF = 8
PIPE = 4

_F32 = jnp.float32
_HIGH = lax.Precision.HIGHEST


def _dot(a, b):
    return jnp.dot(a, b, precision=_HIGH, preferred_element_type=_F32)


def _deg_body(dst_hbm, out_hbm, didx, hist):
    cid = lax.axis_index("c")
    sid = lax.axis_index("s")
    wid = sid * NC + cid
    pltpu.sync_copy(dst_hbm.at[wid], didx)

    def zero(i, c):
        hist[pl.ds(i * LANES, LANES)] = jnp.zeros((LANES,), _F32)
        return c
    lax.fori_loop(0, N // LANES, zero, 0)

    ones = jnp.ones((LANES,), _F32)

    def accum(i, c):
        v = didx[pl.ds(i * LANES, LANES)]
        plsc.addupdate_scatter(hist, [v], ones)
        return c
    lax.fori_loop(0, EPW // LANES, accum, 0)
    pltpu.sync_copy(hist, out_hbm.at[wid])


_deg_call = pl.kernel(
    _deg_body,
    out_type=jax.ShapeDtypeStruct((NW, N), _F32),
    compiler_params=pltpu.CompilerParams(needs_layout_passes=False,
                                        use_tc_tiling_on_sc=False),
    mesh=plsc.VectorSubcoreMesh(core_axis_name="c", subcore_axis_name="s",
                                num_cores=NC, num_subcores=NS),
    scratch_types=[
        pltpu.VMEM((EPW,), jnp.int32),
        pltpu.VMEM((N,), _F32),
    ],
)


def _edge_body(hp_hbm, src_hbm, dst_hbm, z_hbm, out_hbm, sidx, didx, rows,
               acc, sem):
    cid = lax.axis_index("c")
    sid = lax.axis_index("s")
    wid = sid * NC + cid
    last = NS - 1
    rb = (NS - 1) * RPT_A

    @pl.when(jnp.logical_and(cid == 0, sid < last))
    def _():
        r0 = pl.multiple_of(sid * RPT_A, 8)
        pltpu.sync_copy(hp_hbm.at[pl.ds(r0, RPT_A)], acc.at[pl.ds(r0, RPT_A)])

    @pl.when(jnp.logical_and(cid == 0, sid == last))
    def _():
        pltpu.sync_copy(hp_hbm.at[pl.ds(rb, RPT_B)], acc.at[pl.ds(rb, RPT_B)])

    @pl.when(jnp.logical_and(cid != 0, sid < last))
    def _():
        r0 = pl.multiple_of(sid * RPT_A, 8)
        pltpu.sync_copy(z_hbm, acc.at[pl.ds(r0, RPT_A)])

    @pl.when(jnp.logical_and(cid != 0, sid == last))
    def _():
        pltpu.sync_copy(z_hbm.at[pl.ds(0, RPT_B)], acc.at[pl.ds(rb, RPT_B)])

    @pl.when(sid == last)
    def _():
        pltpu.sync_copy(z_hbm.at[pl.ds(0, NPAD - N)], acc.at[pl.ds(N, NPAD - N)])

    pltpu.sync_copy(src_hbm.at[wid], sidx)
    pltpu.sync_copy(dst_hbm.at[wid], didx)
    plsc.subcore_barrier()

    def g_start(j, b):
        pltpu.async_copy(hp_hbm.at[sidx.at[j]], rows.at[b], sem.at[b])

    def g_wait(b):
        pltpu.make_async_copy(hp_hbm.at[sidx.at[0]], rows.at[b],
                              sem.at[b]).wait()

    def s_start(j, b):
        pltpu.async_copy(rows.at[b], acc.at[didx.at[j]], sem.at[b], add=True)

    def s_wait(b):
        pltpu.make_async_copy(rows.at[b], acc.at[didx.at[0]],
                              sem.at[b]).wait()

    for _p in range(PIPE):
        g_start(_p, _p)

    def step(j, c):
        b = lax.rem(j, NBUF)
        bn = lax.rem(j + PIPE, NBUF)
        g_wait(b)
        s_start(j, b)

        @pl.when(j >= PIPE)
        def _():
            s_wait(bn)

        @pl.when(j + PIPE < NCHUNK)
        def _():
            g_start(j + PIPE, bn)
        return c
    lax.fori_loop(0, NCHUNK, step, 0)
    for _p in range(PIPE):
        s_wait((NCHUNK - PIPE + _p) % NBUF)

    plsc.subcore_barrier()

    @pl.when(sid < last)
    def _():
        r0 = pl.multiple_of(sid * RPT_A, 8)
        pltpu.sync_copy(acc.at[pl.ds(r0, RPT_A)],
                        out_hbm.at[cid, pl.ds(r0, RPT_A)])

    @pl.when(sid == last)
    def _():
        pltpu.sync_copy(acc.at[pl.ds(rb, RPT_B)],
                        out_hbm.at[cid, pl.ds(rb, RPT_B)])


_edge_call = pl.kernel(
    _edge_body,
    out_type=jax.ShapeDtypeStruct((NC, N, H), _F32),
    compiler_params=pltpu.CompilerParams(needs_layout_passes=False,
                                        use_tc_tiling_on_sc=False),
    mesh=plsc.VectorSubcoreMesh(core_axis_name="c", subcore_axis_name="s",
                                num_cores=NC, num_subcores=NS),
    scratch_types=[
        pltpu.VMEM((NCHUNK, K), jnp.int32),
        pltpu.VMEM((NCHUNK, K), jnp.int32),
        pltpu.VMEM((NBUF, K, H), _F32),
        pltpu.VMEM_SHARED((NPAD, H), _F32),
        pltpu.SemaphoreType.DMA((NBUF,)),
    ],
)


def _mm0_body(x_ref, w_ref, hw_ref):
    hw_ref[...] = _dot(x_ref[...], w_ref[...])


_mm0_call = pl.pallas_call(
    _mm0_body,
    out_shape=jax.ShapeDtypeStruct((N, H), _F32),
)


def _head_body(degp_ref, hw_ref, hp_ref, dinv_ref):
    deg = jnp.sum(degp_ref[...], axis=0) + 1.0
    dinv = lax.rsqrt(deg)[:, None]
    hp_ref[...] = dinv * hw_ref[...]
    dinv_ref[...] = dinv


_head_call = pl.pallas_call(
    _head_body,
    out_shape=(jax.ShapeDtypeStruct((N, H), _F32),
               jax.ShapeDtypeStruct((N, 1), _F32)),
)


def _bn_relu(raw_ref, dinv_ref, b_ref, g_ref, be_ref):
    dinv = dinv_ref[...]
    y = dinv * (raw_ref[0] + raw_ref[1]) + b_ref[...]
    m = jnp.mean(y, axis=0, keepdims=True)
    cdev = y - m
    v = jnp.mean(cdev * cdev, axis=0, keepdims=True)
    hbn = cdev * lax.rsqrt(v + 1e-5) * g_ref[...] + be_ref[...]
    return jnp.maximum(hbn, 0.0)


def _mid_body(raw_ref, dinv_ref, b_ref, g_ref, be_ref, w_ref, out_ref):
    h = _bn_relu(raw_ref, dinv_ref, b_ref, g_ref, be_ref)
    out_ref[...] = dinv_ref[...] * _dot(h, w_ref[...])


_mid_call = pl.pallas_call(
    _mid_body,
    out_shape=jax.ShapeDtypeStruct((N, H), _F32),
)


def _final_body(raw_ref, dinv_ref, b_ref, g_ref, be_ref, batch_ref,
                f1w_ref, f1b_ref, f2w_ref, f2b_ref, out_ref):
    h = _bn_relu(raw_ref, dinv_ref, b_ref, g_ref, be_ref)
    gids = lax.broadcasted_iota(jnp.int32, (G, N), 0)
    mask = (batch_ref[...] == gids).astype(_F32)
    cnt = jnp.sum(mask, axis=1, keepdims=True)
    pooled = _dot(mask, h) / jnp.maximum(cnt, 1.0)
    z = jnp.maximum(_dot(pooled, f1w_ref[...]) + f1b_ref[...], 0.0)
    z2 = _dot(z, f2w_ref[...]) + f2b_ref[...]
    mx = jnp.max(z2, axis=1, keepdims=True)
    lse = jnp.log(jnp.sum(jnp.exp(z2 - mx), axis=1, keepdims=True)) + mx
    out_ref[...] = z2 - lse


_final_call = pl.pallas_call(
    _final_body,
    out_shape=jax.ShapeDtypeStruct((G, C), _F32),
)


def kernel(x, edge_index, batch, W0, b0, g0, be0, W1, b1, g1, be1,
           W2, b2, g2, be2, fc1_w, fc1_b, fc2_w, fc2_b):
    ei = edge_index.astype(jnp.int32)
    npad_e = NW * EPWP - E
    src = jnp.concatenate(
        [ei[0], jnp.zeros((npad_e,), jnp.int32)]).reshape(NW, NCHUNK, K)
    dst = jnp.concatenate(
        [ei[1], N + (jnp.arange(npad_e, dtype=jnp.int32) % (NPAD - N))]
    ).reshape(NW, NCHUNK, K)
    dst_flat = ei[1].reshape(NW, EPW)
    zrows = jnp.zeros((RPT_A, H), _F32)
    batch2d = batch.astype(jnp.int32).reshape(1, N)

    degp = _deg_call(dst_flat)
    hw0 = _mm0_call(x, W0)
    hp, dinv = _head_call(degp, hw0)
    raw = _edge_call(hp, src, dst, zrows)
    hp = _mid_call(raw, dinv, b0.reshape(1, H), g0.reshape(1, H),
                   be0.reshape(1, H), W1)
    raw = _edge_call(hp, src, dst, zrows)
    hp = _mid_call(raw, dinv, b1.reshape(1, H), g1.reshape(1, H),
                   be1.reshape(1, H), W2)
    raw = _edge_call(hp, src, dst, zrows)
    return _final_call(raw, dinv, b2.reshape(1, H), g2.reshape(1, H),
                       be2.reshape(1, H), batch2d, fc1_w,
                       fc1_b.reshape(1, H // 2), fc2_w, fc2_b.reshape(1, C))


# speedup vs baseline: 1.7112x; 1.7112x over previous
import jax
import jax.numpy as jnp
from jax import lax
from jax.experimental import pallas as pl
from jax.experimental.pallas import tpu as pltpu
from jax.experimental.pallas import tpu_sc as plsc

N = 10000
E = 320000
F_IN = 128
H = 64
C = 10
G = 64

NC = 2
NS = 16
NW = NC * NS
EPW = E // NW
K = 80
NCHUNK = -(-EPW // K)
EPWP = NCHUNK * K
NPAD = N + 16
RPT_A = 640
RPT_B = N - (NS - 1) * RPT_A
LANES = 16
NBUF = 8
PIPE = 4

_F32 = jnp.float32
_HIGH = lax.Precision.HIGHEST


def _dot(a, b):
    return jnp.dot(a, b, precision=_HIGH, preferred_element_type=_F32)


def _deg_body(dst_hbm, out_hbm, didx, hist):
    cid = lax.axis_index("c")
    sid = lax.axis_index("s")
    wid = sid * NC + cid
    pltpu.sync_copy(dst_hbm.at[wid], didx)

    def zero(i, c):
        hist[pl.ds(i * LANES, LANES)] = jnp.zeros((LANES,), _F32)
        return c
    lax.fori_loop(0, N // LANES, zero, 0)

    ones = jnp.ones((LANES,), _F32)

    def accum(i, c):
        v = didx[pl.ds(i * LANES, LANES)]
        plsc.addupdate_scatter(hist, [v], ones)
        return c
    lax.fori_loop(0, EPW // LANES, accum, 0)
    pltpu.sync_copy(hist, out_hbm.at[wid])


_deg_call = pl.kernel(
    _deg_body,
    out_type=jax.ShapeDtypeStruct((NW, N), _F32),
    compiler_params=pltpu.CompilerParams(needs_layout_passes=False,
                                        use_tc_tiling_on_sc=False),
    mesh=plsc.VectorSubcoreMesh(core_axis_name="c", subcore_axis_name="s",
                                num_cores=NC, num_subcores=NS),
    scratch_types=[
        pltpu.VMEM((EPW,), jnp.int32),
        pltpu.VMEM((N,), _F32),
    ],
)


def _edge_body(hp_hbm, src_hbm, dst_hbm, z_hbm, out_hbm, sidx, didx, rows,
               acc, sem):
    cid = lax.axis_index("c")
    sid = lax.axis_index("s")
    wid = sid * NC + cid
    last = NS - 1
    rb = (NS - 1) * RPT_A

    @pl.when(jnp.logical_and(cid == 0, sid < last))
    def _():
        r0 = pl.multiple_of(sid * RPT_A, 8)
        pltpu.sync_copy(hp_hbm.at[pl.ds(r0, RPT_A)], acc.at[pl.ds(r0, RPT_A)])

    @pl.when(jnp.logical_and(cid == 0, sid == last))
    def _():
        pltpu.sync_copy(hp_hbm.at[pl.ds(rb, RPT_B)], acc.at[pl.ds(rb, RPT_B)])

    @pl.when(jnp.logical_and(cid != 0, sid < last))
    def _():
        r0 = pl.multiple_of(sid * RPT_A, 8)
        pltpu.sync_copy(z_hbm, acc.at[pl.ds(r0, RPT_A)])

    @pl.when(jnp.logical_and(cid != 0, sid == last))
    def _():
        pltpu.sync_copy(z_hbm.at[pl.ds(0, RPT_B)], acc.at[pl.ds(rb, RPT_B)])

    @pl.when(sid == last)
    def _():
        pltpu.sync_copy(z_hbm.at[pl.ds(0, NPAD - N)], acc.at[pl.ds(N, NPAD - N)])

    pltpu.sync_copy(src_hbm.at[wid], sidx)
    pltpu.sync_copy(dst_hbm.at[wid], didx)
    plsc.subcore_barrier()

    def g_start(j, b):
        pltpu.async_copy(hp_hbm.at[sidx.at[j]], rows.at[b], sem.at[b])

    def g_wait(b):
        pltpu.make_async_copy(hp_hbm.at[sidx.at[0]], rows.at[b],
                              sem.at[b]).wait()

    def s_start(j, b):
        pltpu.async_copy(rows.at[b], acc.at[didx.at[j]], sem.at[b], add=True)

    def s_wait(b):
        pltpu.make_async_copy(rows.at[b], acc.at[didx.at[0]],
                              sem.at[b]).wait()

    for _p in range(PIPE):
        g_start(_p, _p)

    def step(j, c):
        b = lax.rem(j, NBUF)
        bn = lax.rem(j + PIPE, NBUF)
        g_wait(b)
        s_start(j, b)

        @pl.when(j >= PIPE)
        def _():
            s_wait(bn)

        @pl.when(j + PIPE < NCHUNK)
        def _():
            g_start(j + PIPE, bn)
        return c
    lax.fori_loop(0, NCHUNK, step, 0)
    for _p in range(PIPE):
        s_wait((NCHUNK - PIPE + _p) % NBUF)

    plsc.subcore_barrier()

    @pl.when(sid < last)
    def _():
        r0 = pl.multiple_of(sid * RPT_A, 8)
        pltpu.sync_copy(acc.at[pl.ds(r0, RPT_A)],
                        out_hbm.at[cid, pl.ds(r0, RPT_A)])

    @pl.when(sid == last)
    def _():
        pltpu.sync_copy(acc.at[pl.ds(rb, RPT_B)],
                        out_hbm.at[cid, pl.ds(rb, RPT_B)])


_edge_call = pl.kernel(
    _edge_body,
    out_type=jax.ShapeDtypeStruct((NC, N, H), _F32),
    compiler_params=pltpu.CompilerParams(needs_layout_passes=False,
                                        use_tc_tiling_on_sc=False),
    mesh=plsc.VectorSubcoreMesh(core_axis_name="c", subcore_axis_name="s",
                                num_cores=NC, num_subcores=NS),
    scratch_types=[
        pltpu.VMEM((NCHUNK, K), jnp.int32),
        pltpu.VMEM((NCHUNK, K), jnp.int32),
        pltpu.VMEM((NBUF, K, H), _F32),
        pltpu.VMEM_SHARED((NPAD, H), _F32),
        pltpu.SemaphoreType.DMA((NBUF,)),
    ],
)


def _mm0_body(x_ref, w_ref, hw_ref):
    hw_ref[...] = _dot(x_ref[...], w_ref[...])


_mm0_call = pl.pallas_call(
    _mm0_body,
    out_shape=jax.ShapeDtypeStruct((N, H), _F32),
)


def _head_body(degp_ref, hw_ref, hp_ref, dinv_ref):
    deg = jnp.sum(degp_ref[...], axis=0) + 1.0
    dinv = lax.rsqrt(deg)[:, None]
    hp_ref[...] = dinv * hw_ref[...]
    dinv_ref[...] = dinv


_head_call = pl.pallas_call(
    _head_body,
    out_shape=(jax.ShapeDtypeStruct((N, H), _F32),
               jax.ShapeDtypeStruct((N, 1), _F32)),
)


def _bn_relu(raw_ref, dinv_ref, b_ref, g_ref, be_ref):
    dinv = dinv_ref[...]
    y = dinv * (raw_ref[0] + raw_ref[1]) + b_ref[...]
    m = jnp.mean(y, axis=0, keepdims=True)
    cdev = y - m
    v = jnp.mean(cdev * cdev, axis=0, keepdims=True)
    hbn = cdev * lax.rsqrt(v + 1e-5) * g_ref[...] + be_ref[...]
    return jnp.maximum(hbn, 0.0)


def _mid_body(raw_ref, dinv_ref, b_ref, g_ref, be_ref, w_ref, out_ref):
    h = _bn_relu(raw_ref, dinv_ref, b_ref, g_ref, be_ref)
    out_ref[...] = dinv_ref[...] * _dot(h, w_ref[...])


_mid_call = pl.pallas_call(
    _mid_body,
    out_shape=jax.ShapeDtypeStruct((N, H), _F32),
)


def _final_body(raw_ref, dinv_ref, b_ref, g_ref, be_ref, batch_ref,
                f1w_ref, f1b_ref, f2w_ref, f2b_ref, out_ref):
    h = _bn_relu(raw_ref, dinv_ref, b_ref, g_ref, be_ref)
    gids = lax.broadcasted_iota(jnp.int32, (G, N), 0)
    mask = (batch_ref[...] == gids).astype(_F32)
    cnt = jnp.sum(mask, axis=1, keepdims=True)
    pooled = _dot(mask, h) / jnp.maximum(cnt, 1.0)
    z = jnp.maximum(_dot(pooled, f1w_ref[...]) + f1b_ref[...], 0.0)
    z2 = _dot(z, f2w_ref[...]) + f2b_ref[...]
    mx = jnp.max(z2, axis=1, keepdims=True)
    lse = jnp.log(jnp.sum(jnp.exp(z2 - mx), axis=1, keepdims=True)) + mx
    out_ref[...] = z2 - lse


_final_call = pl.pallas_call(
    _final_body,
    out_shape=jax.ShapeDtypeStruct((G, C), _F32),
)


def kernel(x, edge_index, batch, W0, b0, g0, be0, W1, b1, g1, be1,
           W2, b2, g2, be2, fc1_w, fc1_b, fc2_w, fc2_b):
    ei = edge_index.astype(jnp.int32)
    npad_e = NW * EPWP - E
    src = jnp.concatenate(
        [ei[0], jnp.zeros((npad_e,), jnp.int32)]).reshape(NW, NCHUNK, K)
    dst = jnp.concatenate(
        [ei[1], N + (jnp.arange(npad_e, dtype=jnp.int32) % (NPAD - N))]
    ).reshape(NW, NCHUNK, K)
    dst_flat = ei[1].reshape(NW, EPW)
    zrows = jnp.zeros((RPT_A, H), _F32)
    batch2d = batch.astype(jnp.int32).reshape(1, N)

    degp = _deg_call(dst_flat)
    hw0 = _mm0_call(x, W0)
    hp, dinv = _head_call(degp, hw0)
    raw = _edge_call(hp, src, dst, zrows)
    hp = _mid_call(raw, dinv, b0.reshape(1, H), g0.reshape(1, H),
                   be0.reshape(1, H), W1)
    raw = _edge_call(hp, src, dst, zrows)
    hp = _mid_call(raw, dinv, b1.reshape(1, H), g1.reshape(1, H),
                   be1.reshape(1, H), W2)
    raw = _edge_call(hp, src, dst, zrows)
    return _final_call(raw, dinv, b2.reshape(1, H), g2.reshape(1, H),
                       be2.reshape(1, H), batch2d, fc1_w,
                       fc1_b.reshape(1, H // 2), fc2_w, fc2_b.reshape(1, C))


# speedup vs baseline: 1.7625x; 1.0300x over previous
import jax
import jax.numpy as jnp
from jax import lax
from jax.experimental import pallas as pl
from jax.experimental.pallas import tpu as pltpu
from jax.experimental.pallas import tpu_sc as plsc

N = 10000
E = 320000
F_IN = 128
H = 64
C = 10
G = 64

NC = 2
NS = 16
NW = NC * NS
EPW = E // NW
K = 80
NCHUNK = -(-EPW // K)
EPWP = NCHUNK * K
NPAD = N + 16
RPT_A = 640
RPT_B = N - (NS - 1) * RPT_A
LANES = 16
NBUF = 8
PIPE = 4

_F32 = jnp.float32
_HIGH = lax.Precision.HIGHEST


def _dot(a, b):
    return jnp.dot(a, b, precision=_HIGH, preferred_element_type=_F32)


def _deg_body(dst_hbm, out_hbm, didx, hist):
    cid = lax.axis_index("c")
    sid = lax.axis_index("s")
    wid = sid * NC + cid
    pltpu.sync_copy(dst_hbm.at[wid], didx)

    def zero(i, c):
        hist[pl.ds(i * LANES, LANES)] = jnp.zeros((LANES,), _F32)
        return c
    lax.fori_loop(0, N // LANES, zero, 0)

    ones = jnp.ones((LANES,), _F32)

    def accum(i, c):
        v = didx[pl.ds(i * LANES, LANES)]
        plsc.addupdate_scatter(hist, [v], ones)
        return c
    lax.fori_loop(0, EPW // LANES, accum, 0)
    pltpu.sync_copy(hist, out_hbm.at[wid])


_deg_call = pl.kernel(
    _deg_body,
    out_type=jax.ShapeDtypeStruct((NW, N), _F32),
    compiler_params=pltpu.CompilerParams(needs_layout_passes=False,
                                        use_tc_tiling_on_sc=False),
    mesh=plsc.VectorSubcoreMesh(core_axis_name="c", subcore_axis_name="s",
                                num_cores=NC, num_subcores=NS),
    scratch_types=[
        pltpu.VMEM((EPW,), jnp.int32),
        pltpu.VMEM((N,), _F32),
    ],
)


def _edge_body(hp_hbm, src_hbm, dst_hbm, z_hbm, out_hbm, sidx, didx, rows,
               acc, sem):
    cid = lax.axis_index("c")
    sid = lax.axis_index("s")
    wid = sid * NC + cid
    last = NS - 1
    rb = (NS - 1) * RPT_A

    @pl.when(jnp.logical_and(cid == 0, sid < last))
    def _():
        r0 = pl.multiple_of(sid * RPT_A, 8)
        pltpu.async_copy(hp_hbm.at[pl.ds(r0, RPT_A)], acc.at[pl.ds(r0, RPT_A)],
                         sem.at[NBUF])

    @pl.when(jnp.logical_and(cid == 0, sid == last))
    def _():
        pltpu.async_copy(hp_hbm.at[pl.ds(rb, RPT_B)], acc.at[pl.ds(rb, RPT_B)],
                         sem.at[NBUF])

    @pl.when(jnp.logical_and(cid != 0, sid < last))
    def _():
        r0 = pl.multiple_of(sid * RPT_A, 8)
        pltpu.async_copy(z_hbm, acc.at[pl.ds(r0, RPT_A)], sem.at[NBUF])

    @pl.when(jnp.logical_and(cid != 0, sid == last))
    def _():
        pltpu.async_copy(z_hbm.at[pl.ds(0, RPT_B)], acc.at[pl.ds(rb, RPT_B)],
                         sem.at[NBUF])

    @pl.when(sid == last)
    def _():
        pltpu.async_copy(z_hbm.at[pl.ds(0, NPAD - N)],
                         acc.at[pl.ds(N, NPAD - N)], sem.at[NBUF + 2])

    pltpu.async_copy(dst_hbm.at[wid], didx, sem.at[NBUF + 1])
    pltpu.sync_copy(src_hbm.at[wid], sidx)

    def g_start(j, b):
        pltpu.async_copy(hp_hbm.at[sidx.at[j]], rows.at[b], sem.at[b])

    def g_wait(b):
        pltpu.make_async_copy(hp_hbm.at[sidx.at[0]], rows.at[b],
                              sem.at[b]).wait()

    def s_start(j, b):
        pltpu.async_copy(rows.at[b], acc.at[didx.at[j]], sem.at[b], add=True)

    def s_wait(b):
        pltpu.make_async_copy(rows.at[b], acc.at[didx.at[0]],
                              sem.at[b]).wait()

    for _p in range(PIPE):
        g_start(_p, _p)

    @pl.when(jnp.logical_and(cid == 0, sid < last))
    def _():
        r0 = pl.multiple_of(sid * RPT_A, 8)
        pltpu.make_async_copy(hp_hbm.at[pl.ds(r0, RPT_A)],
                              acc.at[pl.ds(r0, RPT_A)], sem.at[NBUF]).wait()

    @pl.when(jnp.logical_and(cid == 0, sid == last))
    def _():
        pltpu.make_async_copy(hp_hbm.at[pl.ds(rb, RPT_B)],
                              acc.at[pl.ds(rb, RPT_B)], sem.at[NBUF]).wait()

    @pl.when(jnp.logical_and(cid != 0, sid < last))
    def _():
        r0 = pl.multiple_of(sid * RPT_A, 8)
        pltpu.make_async_copy(z_hbm, acc.at[pl.ds(r0, RPT_A)],
                              sem.at[NBUF]).wait()

    @pl.when(jnp.logical_and(cid != 0, sid == last))
    def _():
        pltpu.make_async_copy(z_hbm.at[pl.ds(0, RPT_B)],
                              acc.at[pl.ds(rb, RPT_B)], sem.at[NBUF]).wait()

    @pl.when(sid == last)
    def _():
        pltpu.make_async_copy(z_hbm.at[pl.ds(0, NPAD - N)],
                              acc.at[pl.ds(N, NPAD - N)],
                              sem.at[NBUF + 2]).wait()

    pltpu.make_async_copy(dst_hbm.at[wid], didx, sem.at[NBUF + 1]).wait()
    plsc.subcore_barrier()

    def step(j, c):
        b = lax.rem(j, NBUF)
        bn = lax.rem(j + PIPE, NBUF)
        g_wait(b)
        s_start(j, b)

        @pl.when(j >= PIPE)
        def _():
            s_wait(bn)

        @pl.when(j + PIPE < NCHUNK)
        def _():
            g_start(j + PIPE, bn)
        return c
    lax.fori_loop(0, NCHUNK, step, 0)
    for _p in range(PIPE):
        s_wait((NCHUNK - PIPE + _p) % NBUF)

    plsc.subcore_barrier()

    @pl.when(sid < last)
    def _():
        r0 = pl.multiple_of(sid * RPT_A, 8)
        pltpu.sync_copy(acc.at[pl.ds(r0, RPT_A)],
                        out_hbm.at[cid, pl.ds(r0, RPT_A)])

    @pl.when(sid == last)
    def _():
        pltpu.sync_copy(acc.at[pl.ds(rb, RPT_B)],
                        out_hbm.at[cid, pl.ds(rb, RPT_B)])


_edge_call = pl.kernel(
    _edge_body,
    out_type=jax.ShapeDtypeStruct((NC, N, H), _F32),
    compiler_params=pltpu.CompilerParams(needs_layout_passes=False,
                                        use_tc_tiling_on_sc=False),
    mesh=plsc.VectorSubcoreMesh(core_axis_name="c", subcore_axis_name="s",
                                num_cores=NC, num_subcores=NS),
    scratch_types=[
        pltpu.VMEM((NCHUNK, K), jnp.int32),
        pltpu.VMEM((NCHUNK, K), jnp.int32),
        pltpu.VMEM((NBUF, K, H), _F32),
        pltpu.VMEM_SHARED((NPAD, H), _F32),
        pltpu.SemaphoreType.DMA((NBUF + 3,)),
    ],
)


def _mm0_body(x_ref, w_ref, hw_ref):
    hw_ref[...] = _dot(x_ref[...], w_ref[...])


_mm0_call = pl.pallas_call(
    _mm0_body,
    out_shape=jax.ShapeDtypeStruct((N, H), _F32),
)


def _head_body(degp_ref, hw_ref, hp_ref, dinv_ref):
    deg = jnp.sum(degp_ref[...], axis=0) + 1.0
    dinv = lax.rsqrt(deg)[:, None]
    hp_ref[...] = dinv * hw_ref[...]
    dinv_ref[...] = dinv


_head_call = pl.pallas_call(
    _head_body,
    out_shape=(jax.ShapeDtypeStruct((N, H), _F32),
               jax.ShapeDtypeStruct((N, 1), _F32)),
)


def _bn_relu(raw_ref, dinv_ref, b_ref, g_ref, be_ref):
    dinv = dinv_ref[...]
    y = dinv * (raw_ref[0] + raw_ref[1]) + b_ref[...]
    m = jnp.mean(y, axis=0, keepdims=True)
    cdev = y - m
    v = jnp.mean(cdev * cdev, axis=0, keepdims=True)
    hbn = cdev * lax.rsqrt(v + 1e-5) * g_ref[...] + be_ref[...]
    return jnp.maximum(hbn, 0.0)


def _mid_body(raw_ref, dinv_ref, b_ref, g_ref, be_ref, w_ref, out_ref):
    h = _bn_relu(raw_ref, dinv_ref, b_ref, g_ref, be_ref)
    out_ref[...] = dinv_ref[...] * _dot(h, w_ref[...])


_mid_call = pl.pallas_call(
    _mid_body,
    out_shape=jax.ShapeDtypeStruct((N, H), _F32),
)


def _final_body(raw_ref, dinv_ref, b_ref, g_ref, be_ref, batch_ref,
                f1w_ref, f1b_ref, f2w_ref, f2b_ref, out_ref):
    h = _bn_relu(raw_ref, dinv_ref, b_ref, g_ref, be_ref)
    gids = lax.broadcasted_iota(jnp.int32, (G, N), 0)
    mask = (batch_ref[...] == gids).astype(_F32)
    cnt = jnp.sum(mask, axis=1, keepdims=True)
    pooled = _dot(mask, h) / jnp.maximum(cnt, 1.0)
    z = jnp.maximum(_dot(pooled, f1w_ref[...]) + f1b_ref[...], 0.0)
    z2 = _dot(z, f2w_ref[...]) + f2b_ref[...]
    mx = jnp.max(z2, axis=1, keepdims=True)
    lse = jnp.log(jnp.sum(jnp.exp(z2 - mx), axis=1, keepdims=True)) + mx
    out_ref[...] = z2 - lse


_final_call = pl.pallas_call(
    _final_body,
    out_shape=jax.ShapeDtypeStruct((G, C), _F32),
)


def kernel(x, edge_index, batch, W0, b0, g0, be0, W1, b1, g1, be1,
           W2, b2, g2, be2, fc1_w, fc1_b, fc2_w, fc2_b):
    ei = edge_index.astype(jnp.int32)
    npad_e = NW * EPWP - E
    src = jnp.concatenate(
        [ei[0], jnp.zeros((npad_e,), jnp.int32)]).reshape(NW, NCHUNK, K)
    dst = jnp.concatenate(
        [ei[1], N + (jnp.arange(npad_e, dtype=jnp.int32) % (NPAD - N))]
    ).reshape(NW, NCHUNK, K)
    dst_flat = ei[1].reshape(NW, EPW)
    zrows = jnp.zeros((RPT_A, H), _F32)
    batch2d = batch.astype(jnp.int32).reshape(1, N)

    degp = _deg_call(dst_flat)
    hw0 = _mm0_call(x, W0)
    hp, dinv = _head_call(degp, hw0)
    raw = _edge_call(hp, src, dst, zrows)
    hp = _mid_call(raw, dinv, b0.reshape(1, H), g0.reshape(1, H),
                   be0.reshape(1, H), W1)
    raw = _edge_call(hp, src, dst, zrows)
    hp = _mid_call(raw, dinv, b1.reshape(1, H), g1.reshape(1, H),
                   be1.reshape(1, H), W2)
    raw = _edge_call(hp, src, dst, zrows)
    return _final_call(raw, dinv, b2.reshape(1, H), g2.reshape(1, H),
                       be2.reshape(1, H), batch2d, fc1_w,
                       fc1_b.reshape(1, H // 2), fc2_w, fc2_b.reshape(1, C))
